# baseline (speedup 1.0000x reference)
import functools

import jax
import jax.numpy as jnp
from jax import lax
from jax.experimental import pallas as pl
from jax.experimental.pallas import tpu as pltpu
from jax.experimental.pallas import tpu_sc as plsc

NUM_CORES = 2
NUM_SUBCORES = 16
NUM_WORKERS = NUM_CORES * NUM_SUBCORES
LANES = 16

NUM_USERS_ = 1000000
NUM_ITEMS_ = 1000000
BATCH = 16384
RANK = 16
PACK = 128 // RANK
WIDTH = 128

BPW = BATCH // NUM_WORKERS
CHUNK = 128
NCHUNK = BPW // CHUNK
GPC = CHUNK // LANES


def _bpr_body(uid_hbm, pid_hbm, nid_hbm, utab_hbm, itab_hbm,
              pos_hbm, neg_hbm,
              uid_v, pid_v, nid_v, ublk_v, pblk_v, nblk_v,
              ubuf_v, pbuf_v, nbuf_v, pos_v, neg_v, sem):
    c = lax.axis_index("c")
    s = lax.axis_index("s")
    wid = s * NUM_CORES + c
    base = wid * BPW

    for t in range(NCHUNK):
        off = base + t * CHUNK
        pltpu.sync_copy(uid_hbm.at[pl.ds(off, CHUNK)], uid_v.at[t])
        pltpu.sync_copy(pid_hbm.at[pl.ds(off, CHUNK)], pid_v.at[t])
        pltpu.sync_copy(nid_hbm.at[pl.ds(off, CHUNK)], nid_v.at[t])
    for t in range(NCHUNK):
        for v in range(GPC):
            sl = pl.ds(v * LANES, LANES)
            ublk_v[t, sl] = lax.shift_right_logical(uid_v[t, sl], 3)
            pblk_v[t, sl] = lax.shift_right_logical(pid_v[t, sl], 3)
            nblk_v[t, sl] = lax.shift_right_logical(nid_v[t, sl], 3)

    iota = lax.iota(jnp.int32, LANES)

    for t in range(NCHUNK):
        cps = (pltpu.async_copy(utab_hbm.at[ublk_v.at[t]], ubuf_v, sem),
               pltpu.async_copy(itab_hbm.at[pblk_v.at[t]], pbuf_v, sem),
               pltpu.async_copy(itab_hbm.at[nblk_v.at[t]], nbuf_v, sem))
        for cp in cps:
            cp.wait()
        for g in range(GPC):
            sl = pl.ds(g * LANES, LANES)
            rows = g * LANES + iota
            ucol = (uid_v[t, sl] & 7) * RANK
            pcol = (pid_v[t, sl] & 7) * RANK
            ncol = (nid_v[t, sl] & 7) * RANK
            accp = jnp.zeros((LANES,), jnp.float32)
            accn = jnp.zeros((LANES,), jnp.float32)
            for j in range(RANK):
                u = plsc.load_gather(ubuf_v, [rows, ucol + j])
                p = plsc.load_gather(pbuf_v, [rows, pcol + j])
                n = plsc.load_gather(nbuf_v, [rows, ncol + j])
                accp = accp + u * p
                accn = accn + u * n
            out = pl.ds(t * CHUNK + g * LANES, LANES)
            pos_v[out] = accp
            neg_v[out] = accn

    pltpu.sync_copy(pos_v, pos_hbm.at[pl.ds(base, BPW)])
    pltpu.sync_copy(neg_v, neg_hbm.at[pl.ds(base, BPW)])


@functools.partial(
    pl.kernel,
    out_type=(jax.ShapeDtypeStruct((BATCH,), jnp.float32),
              jax.ShapeDtypeStruct((BATCH,), jnp.float32)),
    mesh=plsc.VectorSubcoreMesh(core_axis_name="c", subcore_axis_name="s"),
    scratch_types=[
        pltpu.VMEM((NCHUNK, CHUNK), jnp.int32),
        pltpu.VMEM((NCHUNK, CHUNK), jnp.int32),
        pltpu.VMEM((NCHUNK, CHUNK), jnp.int32),
        pltpu.VMEM((NCHUNK, CHUNK), jnp.int32),
        pltpu.VMEM((NCHUNK, CHUNK), jnp.int32),
        pltpu.VMEM((NCHUNK, CHUNK), jnp.int32),
        pltpu.VMEM((CHUNK, WIDTH), jnp.float32),
        pltpu.VMEM((CHUNK, WIDTH), jnp.float32),
        pltpu.VMEM((CHUNK, WIDTH), jnp.float32),
        pltpu.VMEM((BPW,), jnp.float32),
        pltpu.VMEM((BPW,), jnp.float32),
        pltpu.SemaphoreType.DMA,
    ],
    compiler_params=pltpu.CompilerParams(needs_layout_passes=False),
)
def _bpr_sc(uid, pid, nid, utab, itab, pos_out, neg_out, *scratch):
    _bpr_body(uid, pid, nid, utab, itab, pos_out, neg_out, *scratch)


def kernel(user_ids, pos_items, neg_items, user_table, item_table):
    utab = user_table.reshape(NUM_USERS_ // PACK, WIDTH)
    itab = item_table.reshape(NUM_ITEMS_ // PACK, WIDTH)
    return _bpr_sc(user_ids.astype(jnp.int32),
                   pos_items.astype(jnp.int32),
                   neg_items.astype(jnp.int32),
                   utab, itab)


# speedup vs baseline: 1.0277x; 1.0277x over previous
import functools

import jax
import jax.numpy as jnp
from jax import lax
from jax.experimental import pallas as pl
from jax.experimental.pallas import tpu as pltpu
from jax.experimental.pallas import tpu_sc as plsc

NUM_CORES = 2
NUM_SUBCORES = 16
NUM_WORKERS = NUM_CORES * NUM_SUBCORES
LANES = 16

BATCH = 16384
RANK = 16

BPW = BATCH // NUM_WORKERS
NGROUP = BPW // LANES


def _bpr_body(uid_hbm, pid_hbm, nid_hbm, utab, itab,
              pos_hbm, neg_hbm,
              uid_v, pid_v, nid_v, ubuf, pbuf, nbuf,
              pos_v, neg_v, sem):
    c = lax.axis_index("c")
    s = lax.axis_index("s")
    wid = s * NUM_CORES + c
    base = wid * BPW

    pltpu.sync_copy(uid_hbm.at[pl.ds(base, BPW)], uid_v)
    pltpu.sync_copy(pid_hbm.at[pl.ds(base, BPW)], pid_v)
    pltpu.sync_copy(nid_hbm.at[pl.ds(base, BPW)], nid_v)

    hu = pltpu.make_async_copy(utab.at[uid_v], ubuf, sem)
    hp = pltpu.make_async_copy(itab.at[pid_v], pbuf, sem)
    hn = pltpu.make_async_copy(itab.at[nid_v], nbuf, sem)
    hu.start()
    hp.start()
    hn.start()
    hu.wait()
    hp.wait()
    hn.wait()

    def group(g, carry):
        rows = g * LANES + lax.iota(jnp.int32, LANES)
        accp = jnp.zeros((LANES,), jnp.float32)
        accn = jnp.zeros((LANES,), jnp.float32)
        for f in range(RANK):
            fidx = jnp.full((LANES,), f, jnp.int32)
            u = plsc.load_gather(ubuf, [rows, fidx])
            p = plsc.load_gather(pbuf, [rows, fidx])
            n = plsc.load_gather(nbuf, [rows, fidx])
            accp = accp + u * p
            accn = accn + u * n
        sl = pl.ds(g * LANES, LANES)
        pos_v[sl] = accp
        neg_v[sl] = accn
        return carry

    lax.fori_loop(0, NGROUP, group, 0)

    pltpu.sync_copy(pos_v, pos_hbm.at[pl.ds(base, BPW)])
    pltpu.sync_copy(neg_v, neg_hbm.at[pl.ds(base, BPW)])


@functools.partial(
    pl.kernel,
    out_type=(jax.ShapeDtypeStruct((BATCH,), jnp.float32),
              jax.ShapeDtypeStruct((BATCH,), jnp.float32)),
    mesh=plsc.VectorSubcoreMesh(core_axis_name="c", subcore_axis_name="s"),
    scratch_types=[
        pltpu.VMEM((BPW,), jnp.int32),
        pltpu.VMEM((BPW,), jnp.int32),
        pltpu.VMEM((BPW,), jnp.int32),
        pltpu.VMEM((BPW, RANK), jnp.float32),
        pltpu.VMEM((BPW, RANK), jnp.float32),
        pltpu.VMEM((BPW, RANK), jnp.float32),
        pltpu.VMEM((BPW,), jnp.float32),
        pltpu.VMEM((BPW,), jnp.float32),
        pltpu.SemaphoreType.DMA,
    ],
    compiler_params=pltpu.CompilerParams(needs_layout_passes=False,
                                         use_tc_tiling_on_sc=False),
)
def _bpr_sc(uid, pid, nid, utab, itab, pos_out, neg_out, *scratch):
    _bpr_body(uid, pid, nid, utab, itab, pos_out, neg_out, *scratch)


def kernel(user_ids, pos_items, neg_items, user_table, item_table):
    return _bpr_sc(user_ids.astype(jnp.int32),
                   pos_items.astype(jnp.int32),
                   neg_items.astype(jnp.int32),
                   user_table, item_table)
